# Initial kernel scaffold; baseline (speedup 1.0000x reference)
import jax
import jax.numpy as jnp
from jax.experimental import pallas as pl


def kernel(x, edge_index, W1, b1, W2, b2):
    raise NotImplementedError("write your pallas kernel here")


# speedup vs baseline: 10.4663x; 10.4663x over previous
import dataclasses
import functools

import jax
import jax.numpy as jnp
from jax import lax
from jax.experimental import pallas as pl
from jax.experimental.pallas import tpu as pltpu
from jax.experimental.pallas import tpu_sc as plsc

NCORE = 2
NSUB = 16
NW = NCORE * NSUB
CHUNK = 128


def _sc_compiler_params():
    cp = pltpu.CompilerParams()
    if "needs_layout_passes" in pltpu.CompilerParams.__dataclass_fields__:
        cp = dataclasses.replace(cp, needs_layout_passes=False)
    return cp


def _vector_mesh():
    return plsc.VectorSubcoreMesh(
        core_axis_name="c", subcore_axis_name="s",
        num_cores=NCORE, num_subcores=NSUB)


def _zero_fill(buf, nrows, ncols):
    z16 = jnp.zeros((16,), jnp.float32)

    @pl.loop(0, nrows)
    def _(i):
        for c in range(ncols // 16):
            buf[i, pl.ds(c * 16, 16)] = z16


def _sc_degree(col3, n_pad):
    t_steps = col3.shape[1]

    @functools.partial(
        pl.kernel,
        out_type=jax.ShapeDtypeStruct((NW, n_pad), jnp.float32),
        mesh=_vector_mesh(),
        scratch_types=[
            pltpu.VMEM((t_steps, CHUNK), jnp.int32),
            pltpu.VMEM((n_pad,), jnp.float32),
        ],
        compiler_params=_sc_compiler_params(),
    )
    def deg_kernel(col_hbm, out_hbm, idx_v, hist):
        cid = lax.axis_index("c")
        sid = lax.axis_index("s")
        wid = sid * NCORE + cid

        z16 = jnp.zeros((16,), jnp.float32)
        one16 = jnp.full((16,), 1.0, jnp.float32)

        @pl.loop(0, n_pad // 16)
        def _(i):
            hist[pl.ds(i * 16, 16)] = z16

        pltpu.sync_copy(col_hbm.at[wid], idx_v)

        @pl.loop(0, t_steps)
        def _(t):
            for j in range(CHUNK // 16):
                idx16 = idx_v[t, pl.ds(j * 16, 16)]
                plsc.addupdate_scatter(hist, [idx16], one16)

        pltpu.sync_copy(hist, out_hbm.at[wid])

    return deg_kernel(col3)


def _sc_aggregate(table, row3, col3):
    n_pad = table.shape[0]
    t_steps = row3.shape[1]
    rows_per_sub = n_pad // NSUB

    @functools.partial(
        pl.kernel,
        out_type=jax.ShapeDtypeStruct((NCORE, n_pad, 128), jnp.float32),
        mesh=_vector_mesh(),
        scratch_types=[
            pltpu.VMEM((t_steps, CHUNK), jnp.int32),
            pltpu.VMEM((t_steps, CHUNK), jnp.int32),
            pltpu.VMEM((CHUNK, 128), jnp.float32),
            pltpu.VMEM_SHARED((n_pad, 128), jnp.float32),
            pltpu.SemaphoreType.DMA,
        ],
    )
    def agg_kernel(tab_hbm, row_hbm, col_hbm, out_hbm,
                   ridx_v, cidx_v, rows_v, acc, sem):
        cid = lax.axis_index("c")
        sid = lax.axis_index("s")
        wid = sid * NCORE + cid

        _zero_fill(rows_v, CHUNK, 128)
        pltpu.sync_copy(row_hbm.at[wid], ridx_v)
        pltpu.sync_copy(col_hbm.at[wid], cidx_v)

        for j in range(rows_per_sub // 128):
            pltpu.sync_copy(rows_v, acc.at[pl.ds(sid * rows_per_sub + j * 128, 128)])
        plsc.subcore_barrier()

        @pl.loop(0, t_steps)
        def _(t):
            pltpu.async_copy(tab_hbm.at[ridx_v.at[t]], rows_v, sem).wait()
            pltpu.sync_copy(rows_v, acc.at[cidx_v.at[t]], add=True)

        plsc.subcore_barrier()
        pltpu.sync_copy(acc.at[pl.ds(sid * rows_per_sub, rows_per_sub)],
                        out_hbm.at[cid].at[pl.ds(sid * rows_per_sub, rows_per_sub)])

    return agg_kernel(table, row3, col3)


def _dis_from_deg(deg_blk):
    d = jnp.sum(deg_blk[...], axis=0)
    dis = jnp.where(d > 0, lax.rsqrt(jnp.maximum(d, 1e-12)), 0.0)
    return dis[:, None]


def _tc_scale_matmul(xp, w, deg2):
    n_pad = xp.shape[0]
    blk = 512
    grid = (n_pad // blk,)

    def body(x_ref, w_ref, deg_ref, o_ref):
        dis = _dis_from_deg(deg_ref)
        t = jnp.dot(x_ref[...], w_ref[...], preferred_element_type=jnp.float32)
        o_ref[...] = t * dis

    return pl.pallas_call(
        body,
        grid=grid,
        in_specs=[
            pl.BlockSpec((blk, 128), lambda i: (i, 0)),
            pl.BlockSpec((128, 128), lambda i: (0, 0)),
            pl.BlockSpec((NW, blk), lambda i: (0, i)),
        ],
        out_specs=pl.BlockSpec((blk, 128), lambda i: (i, 0)),
        out_shape=jax.ShapeDtypeStruct((n_pad, 128), jnp.float32),
    )(xp, w, deg2)


def _tc_mid(acc2, deg2, b, w):
    n_pad = acc2.shape[1]
    blk = 512
    grid = (n_pad // blk,)

    def body(a_ref, deg_ref, b_ref, w_ref, o_ref):
        dis = _dis_from_deg(deg_ref)
        g = a_ref[0] + a_ref[1]
        h = g * dis + b_ref[...]
        t = jnp.dot(h, w_ref[...], preferred_element_type=jnp.float32)
        o_ref[...] = t * dis

    return pl.pallas_call(
        body,
        grid=grid,
        in_specs=[
            pl.BlockSpec((2, blk, 128), lambda i: (0, i, 0)),
            pl.BlockSpec((NW, blk), lambda i: (0, i)),
            pl.BlockSpec((1, 128), lambda i: (0, 0)),
            pl.BlockSpec((128, 128), lambda i: (0, 0)),
        ],
        out_specs=pl.BlockSpec((blk, 128), lambda i: (i, 0)),
        out_shape=jax.ShapeDtypeStruct((n_pad, 128), jnp.float32),
    )(acc2, deg2, b, w)


def _tc_final(acc2, deg2, b, xp):
    n_pad = xp.shape[0]
    blk = 512
    grid = (n_pad // blk,)

    def body(a_ref, deg_ref, b_ref, x_ref, o_ref):
        dis = _dis_from_deg(deg_ref)
        g = a_ref[0] + a_ref[1]
        h = g * dis + b_ref[...]
        nrm = jnp.sqrt(jnp.sum(h * h, axis=1, keepdims=True))
        o_ref[...] = h / jnp.maximum(nrm, 1e-12) + x_ref[...]

    return pl.pallas_call(
        body,
        grid=grid,
        in_specs=[
            pl.BlockSpec((2, blk, 128), lambda i: (0, i, 0)),
            pl.BlockSpec((NW, blk), lambda i: (0, i)),
            pl.BlockSpec((1, 128), lambda i: (0, 0)),
            pl.BlockSpec((blk, 128), lambda i: (i, 0)),
        ],
        out_specs=pl.BlockSpec((blk, 128), lambda i: (i, 0)),
        out_shape=jax.ShapeDtypeStruct((n_pad, 128), jnp.float32),
    )(acc2, deg2, b, xp)


def kernel(x, edge_index, W1, b1, W2, b2):
    n, d = x.shape
    e = edge_index.shape[1]

    n_pad = ((n + 1 + 2047) // 2048) * 2048
    per_w = -(-e // (NW * CHUNK)) * CHUNK
    e_pad = per_w * NW

    row = edge_index[0]
    col = edge_index[1]
    pad_e = jnp.full((e_pad - e,), n, jnp.int32)
    row3 = jnp.concatenate([row, pad_e]).reshape(NW, per_w // CHUNK, CHUNK)
    col3 = jnp.concatenate([col, pad_e]).reshape(NW, per_w // CHUNK, CHUNK)
    xp = jnp.pad(x, ((0, n_pad - n), (0, 0)))
    b1r = b1.reshape(1, d)
    b2r = b2.reshape(1, d)

    deg2 = _sc_degree(col3, n_pad)
    t1 = _tc_scale_matmul(xp, W1, deg2)
    a1 = _sc_aggregate(t1, row3, col3)
    t2 = _tc_mid(a1, deg2, b1r, W2)
    a2 = _sc_aggregate(t2, row3, col3)
    return _tc_final(a2, deg2, b2r, xp)[:n]
